# baseline (speedup 1.0000x reference)
import jax
import jax.numpy as jnp
from jax.experimental import pallas as pl
from jax.experimental.pallas import tpu as pltpu

_P = 2


def _body(x_ref, w1_ref, nb1_ref, w2t_ref, corr_ref, o_ref):
    h = jnp.dot(x_ref[0], w1_ref[...], preferred_element_type=jnp.float32)
    z = jnp.maximum(h, nb1_ref[...])
    v = z * w2t_ref[...]
    o_ref[...] = jnp.sum(v).reshape(1, 1, 1) + corr_ref[...]


def kernel(embs, W1, b1, W2, b2):
    B, L, D = embs.shape
    H = W1.shape[1]
    w2row = W2.reshape(H)
    x = embs.reshape(B, L // _P, _P * D)
    wbig = jnp.kron(jnp.eye(_P, dtype=W1.dtype), W1)
    nb1big = jnp.tile(-b1, _P).reshape(1, _P * H)
    w2big = jnp.tile(w2row, _P).reshape(1, _P * H)
    corr = (L * (jnp.dot(b1, w2row) + b2[0])).reshape(1, 1)

    out = pl.pallas_call(
        _body,
        grid=(B,),
        in_specs=[
            pl.BlockSpec((1, L // _P, _P * D), lambda i: (i, 0, 0)),
            pl.BlockSpec((_P * D, _P * H), lambda i: (0, 0)),
            pl.BlockSpec((1, _P * H), lambda i: (0, 0)),
            pl.BlockSpec((1, _P * H), lambda i: (0, 0)),
            pl.BlockSpec((1, 1), lambda i: (0, 0)),
        ],
        out_specs=pl.BlockSpec((1, 1, 1), lambda i: (i, 0, 0)),
        out_shape=jax.ShapeDtypeStruct((B, 1, 1), jnp.float32),
    )(x, wbig, nb1big, w2big, corr)
    return out.reshape(B)


# speedup vs baseline: 1.6582x; 1.6582x over previous
import jax
import jax.numpy as jnp
from jax.experimental import pallas as pl
from jax.experimental.pallas import tpu as pltpu

_NBUF = 6


def _body(x_hbm, w1_ref, nb1_ref, w2t_ref, corr_ref, o_ref, buf, sems):
    nchunks = o_ref.shape[0]

    def start(j):
        pltpu.make_async_copy(
            x_hbm.at[j], buf.at[j % _NBUF], sems.at[j % _NBUF]
        ).start()

    for j0 in range(_NBUF):
        start(j0)

    w1 = w1_ref[...]
    nb1 = nb1_ref[...]
    w2t = w2t_ref[...]
    corr = corr_ref[...]
    for i in range(nchunks):
        slot = i % _NBUF
        pltpu.make_async_copy(
            x_hbm.at[i], buf.at[slot], sems.at[slot]
        ).wait()
        h = jnp.dot(buf[slot], w1, preferred_element_type=jnp.float32)
        z = jnp.maximum(h, nb1)
        v = z * w2t
        o_ref[i : i + 1, :] = jnp.sum(v).reshape(1, 1) + corr
        if i + _NBUF < nchunks:
            start(i + _NBUF)


def kernel(embs, W1, b1, W2, b2):
    B, L, D = embs.shape
    H = W1.shape[1]
    w2row = W2.reshape(H)
    corr = (L * (jnp.dot(b1, w2row) + b2[0])).reshape(1, 1)

    out = pl.pallas_call(
        _body,
        in_specs=[
            pl.BlockSpec(memory_space=pltpu.MemorySpace.HBM),
            pl.BlockSpec(memory_space=pltpu.MemorySpace.VMEM),
            pl.BlockSpec(memory_space=pltpu.MemorySpace.VMEM),
            pl.BlockSpec(memory_space=pltpu.MemorySpace.VMEM),
            pl.BlockSpec(memory_space=pltpu.MemorySpace.VMEM),
        ],
        out_specs=pl.BlockSpec(memory_space=pltpu.MemorySpace.VMEM),
        out_shape=jax.ShapeDtypeStruct((B, 1), jnp.float32),
        scratch_shapes=[
            pltpu.VMEM((_NBUF, L, D), jnp.float32),
            pltpu.SemaphoreType.DMA((_NBUF,)),
        ],
    )(embs, W1, (-b1).reshape(1, H), w2row.reshape(1, H), corr)
    return out.reshape(B)


# speedup vs baseline: 1.7115x; 1.0322x over previous
import jax
import jax.numpy as jnp
from jax.experimental import pallas as pl
from jax.experimental.pallas import tpu as pltpu

_NBUF = 4
_SPC = 2


def _body(x_hbm, w1_ref, nb1_ref, w2t_ref, corr_ref, o_ref, buf, sems):
    nchunks = o_ref.shape[0] // _SPC
    L = x_hbm.shape[1]

    def start(j):
        pltpu.make_async_copy(
            x_hbm.at[pl.ds(j * _SPC, _SPC)], buf.at[j % _NBUF], sems.at[j % _NBUF]
        ).start()

    for j0 in range(min(_NBUF, nchunks)):
        start(j0)

    w1 = w1_ref[...]
    nb1 = nb1_ref[...]
    w2t = w2t_ref[...]
    corr = corr_ref[...]
    for i in range(nchunks):
        slot = i % _NBUF
        pltpu.make_async_copy(
            x_hbm.at[pl.ds(i * _SPC, _SPC)], buf.at[slot], sems.at[slot]
        ).wait()
        x = buf[slot].reshape(_SPC * L, w1.shape[0])
        h = jnp.dot(x, w1, preferred_element_type=jnp.float32)
        z = jnp.maximum(h, nb1)
        v = z * w2t
        s = jnp.sum(v.reshape(_SPC, L, v.shape[-1]), axis=(1, 2))
        o_ref[i * _SPC : (i + 1) * _SPC, :] = s.reshape(_SPC, 1) + corr
        if i + _NBUF < nchunks:
            start(i + _NBUF)


def kernel(embs, W1, b1, W2, b2):
    B, L, D = embs.shape
    H = W1.shape[1]
    w2row = W2.reshape(H)
    corr = (L * (jnp.dot(b1, w2row) + b2[0])).reshape(1, 1)

    out = pl.pallas_call(
        _body,
        in_specs=[
            pl.BlockSpec(memory_space=pltpu.MemorySpace.HBM),
            pl.BlockSpec(memory_space=pltpu.MemorySpace.VMEM),
            pl.BlockSpec(memory_space=pltpu.MemorySpace.VMEM),
            pl.BlockSpec(memory_space=pltpu.MemorySpace.VMEM),
            pl.BlockSpec(memory_space=pltpu.MemorySpace.VMEM),
        ],
        out_specs=pl.BlockSpec(memory_space=pltpu.MemorySpace.VMEM),
        out_shape=jax.ShapeDtypeStruct((B, 1), jnp.float32),
        scratch_shapes=[
            pltpu.VMEM((_NBUF, _SPC, L, D), jnp.float32),
            pltpu.SemaphoreType.DMA((_NBUF,)),
        ],
    )(embs, W1, (-b1).reshape(1, H), w2row.reshape(1, H), corr)
    return out.reshape(B)
